# baseline (speedup 1.0000x reference)
import functools

import jax
import jax.numpy as jnp
from jax import lax
from jax.experimental import pallas as pl
from jax.experimental.pallas import tpu as pltpu
from jax.experimental.pallas import tpu_sc as plsc

NC = 2
NS = 16
CHUNK = 128
DEG_W = 16


def _make_deg_kernel(E, N):
    E_sc = E // NC
    n_chunks_sc = E_sc // CHUNK
    base_chunks = n_chunks_sc // NS
    extra = n_chunks_sc % NS
    rows_tile = -(-(N // NS) // 8) * 8
    Np = rows_tile * NS
    mesh = plsc.VectorSubcoreMesh(core_axis_name="c", subcore_axis_name="s")

    @functools.partial(
        pl.kernel,
        out_type=jax.ShapeDtypeStruct((NC * Np, DEG_W), jnp.float32),
        mesh=mesh,
        scratch_types=[
            pltpu.VMEM_SHARED((Np, DEG_W), jnp.float32),
            pltpu.VMEM((CHUNK, DEG_W), jnp.float32),
            pltpu.VMEM((rows_tile, DEG_W), jnp.float32),
            pltpu.VMEM((CHUNK,), jnp.int32),
        ],
    )
    def deg_kernel(dst_hbm, out_hbm, acc, ones_v, zero_v, idx_v):
        c = lax.axis_index("c")
        s = lax.axis_index("s")

        def fill_ones(i, _):
            ones_v[i, :] = jnp.ones((DEG_W,), jnp.float32)
            return 0
        lax.fori_loop(0, CHUNK, fill_ones, 0)

        def fill_zero(i, _):
            zero_v[i, :] = jnp.zeros((DEG_W,), jnp.float32)
            return 0
        lax.fori_loop(0, rows_tile, fill_zero, 0)
        pltpu.sync_copy(zero_v, acc.at[pl.ds(s * rows_tile, rows_tile)])
        plsc.subcore_barrier()

        n_i = base_chunks + jnp.where(s < extra, 1, 0)

        def body(i, _):
            chunk = s + i * NS
            base = c * E_sc + chunk * CHUNK
            pltpu.sync_copy(dst_hbm.at[pl.ds(base, CHUNK)], idx_v)
            pltpu.sync_copy(ones_v, acc.at[idx_v], add=True)
            return 0
        lax.fori_loop(0, n_i, body, 0)
        plsc.subcore_barrier()

        pltpu.sync_copy(acc.at[pl.ds(s * rows_tile, rows_tile)],
                        out_hbm.at[pl.ds(c * Np + s * rows_tile, rows_tile)])

    return deg_kernel


def _make_agg_kernel(E, N, D):
    E_sc = E // NC
    n_chunks_sc = E_sc // CHUNK
    base_chunks = n_chunks_sc // NS
    extra = n_chunks_sc % NS
    rows_tile = -(-(N // NS) // 8) * 8
    Np = rows_tile * NS
    mesh = plsc.VectorSubcoreMesh(core_axis_name="c", subcore_axis_name="s")

    @functools.partial(
        pl.kernel,
        out_type=jax.ShapeDtypeStruct((NC * Np, D), jnp.float32),
        mesh=mesh,
        scratch_types=[
            pltpu.VMEM_SHARED((Np, D), jnp.float32),
            pltpu.VMEM((CHUNK, D), jnp.float32),
            pltpu.VMEM((CHUNK,), jnp.int32),
            pltpu.VMEM((CHUNK,), jnp.int32),
            pltpu.SemaphoreType.DMA,
        ],
    )
    def agg_kernel(g_hbm, src_hbm, dst_hbm, out_hbm, acc, rows_v, src_v,
                   dst_v, sem):
        c = lax.axis_index("c")
        s = lax.axis_index("s")

        def fill_zero(i, _):
            def fill_lane(j, _):
                rows_v[i, pl.ds(j * 16, 16)] = jnp.zeros((16,), jnp.float32)
                return 0
            lax.fori_loop(0, D // 16, fill_lane, 0)
            return 0
        lax.fori_loop(0, CHUNK, fill_zero, 0)
        off = 0
        while off < rows_tile:
            zr = min(CHUNK, rows_tile - off)
            pltpu.sync_copy(
                rows_v.at[pl.ds(0, zr)],
                acc.at[pl.ds(s * rows_tile + off, zr)])
            off += zr
        plsc.subcore_barrier()

        n_i = base_chunks + jnp.where(s < extra, 1, 0)

        def body(i, _):
            chunk = s + i * NS
            base = c * E_sc + chunk * CHUNK
            pltpu.sync_copy(src_hbm.at[pl.ds(base, CHUNK)], src_v)
            pltpu.sync_copy(dst_hbm.at[pl.ds(base, CHUNK)], dst_v)
            pltpu.async_copy(g_hbm.at[src_v], rows_v, sem).wait()
            pltpu.sync_copy(rows_v, acc.at[dst_v], add=True)
            return 0
        lax.fori_loop(0, n_i, body, 0)
        plsc.subcore_barrier()

        pltpu.sync_copy(acc.at[pl.ds(s * rows_tile, rows_tile)],
                        out_hbm.at[pl.ds(c * Np + s * rows_tile, rows_tile)])

    return agg_kernel


def _dot(a, w):
    return lax.dot_general(a, w, (((1,), (0,)), ((), ())),
                           precision=lax.Precision.HIGHEST,
                           preferred_element_type=jnp.float32)


def _matmul_body(x_ref, w_ref, o_ref):
    o_ref[...] = _dot(x_ref[...], w_ref[...])


def _make_scale_body(N, Np):
    def body(deg_ref, h_ref, g_ref, dinv_ref):
        d = 1.0 + deg_ref[0:N, :] + deg_ref[Np:Np + N, :]
        dinv = lax.rsqrt(d)
        dinv_ref[...] = dinv
        g_ref[...] = h_ref[...] * dinv[:, 0:1]
    return body


def _make_mid_body(N, Np):
    def body(a_ref, g_ref, dinv_ref, b_ref, w_ref, o_ref):
        dinv = dinv_ref[...][:, 0:1]
        agg = a_ref[0:N, :] + a_ref[Np:Np + N, :] + g_ref[...]
        xn = jnp.maximum(agg * dinv + b_ref[...], 0.0)
        o_ref[...] = _dot(xn, w_ref[...]) * dinv
    return body


def _make_final_body(N, Np, D_out):
    def body(a_ref, g_ref, dinv_ref, b_ref, o_ref):
        dinv = dinv_ref[...][:, 0:1]
        agg = (a_ref[0:N, 0:D_out] + a_ref[Np:Np + N, 0:D_out]
               + g_ref[0:N, 0:D_out])
        o_ref[...] = agg * dinv + b_ref[...]
    return body


def kernel(x, edge_index, W1, b1, W2, b2, W3, b3):
    N, _ = x.shape
    E0 = edge_index.shape[1]
    grp = NC * NS * CHUNK
    E = -(-E0 // grp) * grp
    pad = E - E0
    rt = -(-(N // NS) // 8) * 8
    Npp = rt * NS
    src_p = jnp.concatenate([edge_index[0], jnp.zeros((pad,), jnp.int32)])
    pad_dst = N + jnp.arange(pad, dtype=jnp.int32) % jnp.int32(Npp - N)
    dst_p = jnp.concatenate([edge_index[1], pad_dst])
    edge_index = jnp.stack([src_p, dst_p])
    D_hid = W1.shape[1]
    D_out = W3.shape[1]
    src = edge_index[0]
    dst = edge_index[1]

    f32 = jnp.float32
    Np = (-(-(N // NS) // 8) * 8) * NS
    deg_parts = _make_deg_kernel(E, N)(dst)

    h1 = pl.pallas_call(
        _matmul_body,
        out_shape=jax.ShapeDtypeStruct((N, D_hid), f32))(x, W1)

    g1, dinv16 = pl.pallas_call(
        _make_scale_body(N, Np),
        out_shape=[jax.ShapeDtypeStruct((N, D_hid), f32),
                   jax.ShapeDtypeStruct((N, DEG_W), f32)])(deg_parts, h1)

    agg_hid = _make_agg_kernel(E, N, D_hid)
    a1 = agg_hid(g1, src, dst)

    g2 = pl.pallas_call(
        _make_mid_body(N, Np),
        out_shape=jax.ShapeDtypeStruct((N, D_hid), f32))(
            a1, g1, dinv16, b1.reshape(1, -1), W2)

    a2 = agg_hid(g2, src, dst)

    W3p = jnp.pad(W3, ((0, 0), (0, D_hid - D_out)))
    g3 = pl.pallas_call(
        _make_mid_body(N, Np),
        out_shape=jax.ShapeDtypeStruct((N, D_hid), f32))(
            a2, g2, dinv16, b2.reshape(1, -1), W3p)

    a3 = agg_hid(g3, src, dst)

    out = pl.pallas_call(
        _make_final_body(N, Np, D_out),
        out_shape=jax.ShapeDtypeStruct((N, D_out), f32))(
            a3, g3, dinv16, b3.reshape(1, -1))

    return out


# speedup vs baseline: 1.4199x; 1.4199x over previous
import functools

import jax
import jax.numpy as jnp
from jax import lax
from jax.experimental import pallas as pl
from jax.experimental.pallas import tpu as pltpu
from jax.experimental.pallas import tpu_sc as plsc

NC = 2
NS = 16
CHUNK = 128
DEG_W = 16


def _make_deg_kernel(E, N):
    E_sc = E // NC
    n_chunks_sc = E_sc // CHUNK
    base_chunks = n_chunks_sc // NS
    extra = n_chunks_sc % NS
    rows_tile = -(-(N // NS) // 8) * 8
    Np = rows_tile * NS
    mesh = plsc.VectorSubcoreMesh(core_axis_name="c", subcore_axis_name="s")

    @functools.partial(
        pl.kernel,
        out_type=jax.ShapeDtypeStruct((NC * Np, DEG_W), jnp.float32),
        mesh=mesh,
        scratch_types=[
            pltpu.VMEM_SHARED((Np, DEG_W), jnp.float32),
            pltpu.VMEM((CHUNK, DEG_W), jnp.float32),
            pltpu.VMEM((rows_tile, DEG_W), jnp.float32),
            pltpu.VMEM((CHUNK,), jnp.int32),
        ],
    )
    def deg_kernel(dst_hbm, out_hbm, acc, ones_v, zero_v, idx_v):
        c = lax.axis_index("c")
        s = lax.axis_index("s")

        def fill_ones(i, _):
            ones_v[i, :] = jnp.ones((DEG_W,), jnp.float32)
            return 0
        lax.fori_loop(0, CHUNK, fill_ones, 0)

        def fill_zero(i, _):
            zero_v[i, :] = jnp.zeros((DEG_W,), jnp.float32)
            return 0
        lax.fori_loop(0, rows_tile, fill_zero, 0)
        pltpu.sync_copy(zero_v, acc.at[pl.ds(s * rows_tile, rows_tile)])
        plsc.subcore_barrier()

        n_i = base_chunks + jnp.where(s < extra, 1, 0)

        def body(i, _):
            chunk = s + i * NS
            base = c * E_sc + chunk * CHUNK
            pltpu.sync_copy(dst_hbm.at[pl.ds(base, CHUNK)], idx_v)
            pltpu.sync_copy(ones_v, acc.at[idx_v], add=True)
            return 0
        lax.fori_loop(0, n_i, body, 0)
        plsc.subcore_barrier()

        pltpu.sync_copy(acc.at[pl.ds(s * rows_tile, rows_tile)],
                        out_hbm.at[pl.ds(c * Np + s * rows_tile, rows_tile)])

    return deg_kernel


def _make_agg_kernel(E, N, D):
    E_sc = E // NC
    n_chunks_sc = E_sc // CHUNK
    base_chunks = n_chunks_sc // NS
    extra = n_chunks_sc % NS
    rows_tile = -(-(N // NS) // 8) * 8
    Np = rows_tile * NS
    mesh = plsc.VectorSubcoreMesh(core_axis_name="c", subcore_axis_name="s")

    @functools.partial(
        pl.kernel,
        out_type=jax.ShapeDtypeStruct((NC * Np, D), jnp.float32),
        mesh=mesh,
        scratch_types=[
            pltpu.VMEM_SHARED((Np, D), jnp.float32),
            pltpu.VMEM((CHUNK, D), jnp.float32),
            pltpu.VMEM((CHUNK,), jnp.int32),
            pltpu.VMEM((CHUNK,), jnp.int32),
            pltpu.SemaphoreType.DMA,
        ],
    )
    def agg_kernel(g_hbm, src_hbm, dst_hbm, out_hbm, acc, rows_v, src_v,
                   dst_v, sem):
        c = lax.axis_index("c")
        s = lax.axis_index("s")

        def fill_zero(i, _):
            def fill_lane(j, _):
                rows_v[i, pl.ds(j * 16, 16)] = jnp.zeros((16,), jnp.float32)
                return 0
            lax.fori_loop(0, D // 16, fill_lane, 0)
            return 0
        lax.fori_loop(0, CHUNK, fill_zero, 0)
        off = 0
        while off < rows_tile:
            zr = min(CHUNK, rows_tile - off)
            pltpu.sync_copy(
                rows_v.at[pl.ds(0, zr)],
                acc.at[pl.ds(s * rows_tile + off, zr)])
            off += zr
        plsc.subcore_barrier()

        n_i = base_chunks + jnp.where(s < extra, 1, 0)

        def body(i, _):
            chunk = s + i * NS
            base = c * E_sc + chunk * CHUNK
            pltpu.sync_copy(src_hbm.at[pl.ds(base, CHUNK)], src_v)
            pltpu.sync_copy(dst_hbm.at[pl.ds(base, CHUNK)], dst_v)
            pltpu.async_copy(g_hbm.at[src_v], rows_v, sem).wait()
            pltpu.sync_copy(rows_v, acc.at[dst_v], add=True)
            return 0
        lax.fori_loop(0, n_i, body, 0)
        plsc.subcore_barrier()

        pltpu.sync_copy(acc.at[pl.ds(s * rows_tile, rows_tile)],
                        out_hbm.at[pl.ds(c * Np + s * rows_tile, rows_tile)])

    return agg_kernel


def _dot(a, w):
    return lax.dot_general(a, w, (((1,), (0,)), ((), ())),
                           precision=lax.Precision.HIGHEST,
                           preferred_element_type=jnp.float32)


def _matmul_body(x_ref, w_ref, o_ref):
    o_ref[...] = _dot(x_ref[...], w_ref[...])


def _make_scale_body(N, Np):
    def body(deg_ref, h_ref, g_ref, dinv_ref):
        d = 1.0 + deg_ref[0:N, :] + deg_ref[Np:Np + N, :]
        dinv = lax.rsqrt(d)
        dinv_ref[...] = dinv
        g_ref[...] = h_ref[...] * dinv[:, 0:1]
    return body


def _make_mid_body(N, Np):
    def body(a_ref, g_ref, dinv_ref, b_ref, w_ref, o_ref):
        dinv = dinv_ref[...][:, 0:1]
        agg = a_ref[0:N, :] + a_ref[Np:Np + N, :] + g_ref[...]
        xn = jnp.maximum(agg * dinv + b_ref[...], 0.0)
        o_ref[...] = _dot(xn, w_ref[...]) * dinv
    return body


def _make_final_body(N, Np, D_out):
    def body(a_ref, g_ref, dinv_ref, b_ref, o_ref):
        dinv = dinv_ref[...][:, 0:1]
        agg = (a_ref[0:N, 0:D_out] + a_ref[Np:Np + N, 0:D_out]
               + g_ref[0:N, 0:D_out])
        o_ref[...] = agg * dinv + b_ref[...]
    return body


def kernel(x, edge_index, W1, b1, W2, b2, W3, b3):
    N, _ = x.shape
    E0 = edge_index.shape[1]
    grp = NC * NS * CHUNK
    E = -(-E0 // grp) * grp
    pad = E - E0
    rt = -(-(N // NS) // 8) * 8
    Npp = rt * NS
    pad_src = jnp.arange(pad, dtype=jnp.int32) % jnp.int32(N)
    src_p = jnp.concatenate([edge_index[0], pad_src])
    pad_dst = N + jnp.arange(pad, dtype=jnp.int32) % jnp.int32(Npp - N)
    dst_p = jnp.concatenate([edge_index[1], pad_dst])
    edge_index = jnp.stack([src_p, dst_p])
    D_hid = W1.shape[1]
    D_out = W3.shape[1]
    src = edge_index[0]
    dst = edge_index[1]

    f32 = jnp.float32
    Np = (-(-(N // NS) // 8) * 8) * NS
    deg_parts = _make_deg_kernel(E, N)(dst)

    h1 = pl.pallas_call(
        _matmul_body,
        out_shape=jax.ShapeDtypeStruct((N, D_hid), f32))(x, W1)

    g1, dinv16 = pl.pallas_call(
        _make_scale_body(N, Np),
        out_shape=[jax.ShapeDtypeStruct((N, D_hid), f32),
                   jax.ShapeDtypeStruct((N, DEG_W), f32)])(deg_parts, h1)

    agg_hid = _make_agg_kernel(E, N, D_hid)
    a1 = agg_hid(g1, src, dst)

    g2 = pl.pallas_call(
        _make_mid_body(N, Np),
        out_shape=jax.ShapeDtypeStruct((N, D_hid), f32))(
            a1, g1, dinv16, b1.reshape(1, -1), W2)

    a2 = agg_hid(g2, src, dst)

    W3p = jnp.pad(W3, ((0, 0), (0, D_hid - D_out)))
    g3 = pl.pallas_call(
        _make_mid_body(N, Np),
        out_shape=jax.ShapeDtypeStruct((N, D_hid), f32))(
            a2, g2, dinv16, b2.reshape(1, -1), W3p)

    a3 = agg_hid(g3, src, dst)

    out = pl.pallas_call(
        _make_final_body(N, Np, D_out),
        out_shape=jax.ShapeDtypeStruct((N, D_out), f32))(
            a3, g3, dinv16, b3.reshape(1, -1))

    return out


# speedup vs baseline: 2.0425x; 1.4385x over previous
import functools

import jax
import jax.numpy as jnp
from jax import lax
from jax.experimental import pallas as pl
from jax.experimental.pallas import tpu as pltpu
from jax.experimental.pallas import tpu_sc as plsc

NC = 2
NS = 16
CHUNK = 128
DEG_W = 16


def _make_deg_kernel(E, N):
    E_sc = E // NC
    n_chunks_sc = E_sc // CHUNK
    base_chunks = n_chunks_sc // NS
    extra = n_chunks_sc % NS
    rows_tile = -(-(N // NS) // 8) * 8
    Np = rows_tile * NS
    mesh = plsc.VectorSubcoreMesh(core_axis_name="c", subcore_axis_name="s")

    @functools.partial(
        pl.kernel,
        out_type=jax.ShapeDtypeStruct((NC * Np, DEG_W), jnp.float32),
        mesh=mesh,
        scratch_types=[
            pltpu.VMEM_SHARED((Np, DEG_W), jnp.float32),
            pltpu.VMEM((CHUNK, DEG_W), jnp.float32),
            pltpu.VMEM((rows_tile, DEG_W), jnp.float32),
            pltpu.VMEM((CHUNK,), jnp.int32),
        ],
    )
    def deg_kernel(dst_hbm, out_hbm, acc, ones_v, zero_v, idx_v):
        c = lax.axis_index("c")
        s = lax.axis_index("s")

        def fill_ones(i, _):
            ones_v[i, :] = jnp.ones((DEG_W,), jnp.float32)
            return 0
        lax.fori_loop(0, CHUNK, fill_ones, 0)

        def fill_zero(i, _):
            zero_v[i, :] = jnp.zeros((DEG_W,), jnp.float32)
            return 0
        lax.fori_loop(0, rows_tile, fill_zero, 0)
        pltpu.sync_copy(zero_v, acc.at[pl.ds(s * rows_tile, rows_tile)])
        plsc.subcore_barrier()

        n_i = base_chunks + jnp.where(s < extra, 1, 0)

        def body(i, _):
            chunk = s + i * NS
            base = c * E_sc + chunk * CHUNK
            pltpu.sync_copy(dst_hbm.at[pl.ds(base, CHUNK)], idx_v)
            pltpu.sync_copy(ones_v, acc.at[idx_v], add=True)
            return 0
        lax.fori_loop(0, n_i, body, 0)
        plsc.subcore_barrier()

        pltpu.sync_copy(acc.at[pl.ds(s * rows_tile, rows_tile)],
                        out_hbm.at[pl.ds(c * Np + s * rows_tile, rows_tile)])

    return deg_kernel


def _make_agg_kernel(E, N, D):
    E_sc = E // NC
    n_chunks_sc = E_sc // CHUNK
    base_chunks = n_chunks_sc // NS
    extra = n_chunks_sc % NS
    rows_tile = -(-(N // NS) // 8) * 8
    Np = rows_tile * NS
    mesh = plsc.VectorSubcoreMesh(core_axis_name="c", subcore_axis_name="s")

    @functools.partial(
        pl.kernel,
        out_type=jax.ShapeDtypeStruct((NC * Np, D), jnp.float32),
        mesh=mesh,
        scratch_types=[
            pltpu.VMEM_SHARED((Np, D), jnp.float32),
            pltpu.VMEM((CHUNK, D), jnp.float32),
            pltpu.VMEM((CHUNK, D), jnp.float32),
            pltpu.VMEM((CHUNK,), jnp.int32),
            pltpu.VMEM((CHUNK,), jnp.int32),
            pltpu.VMEM((CHUNK,), jnp.int32),
            pltpu.VMEM((CHUNK,), jnp.int32),
            pltpu.SemaphoreType.DMA,
            pltpu.SemaphoreType.DMA,
        ],
    )
    def agg_kernel(g_hbm, src_hbm, dst_hbm, out_hbm, acc, rows_a, rows_b,
                   src_a, src_b, dst_a, dst_b, gsa, gsb):
        c = lax.axis_index("c")
        s = lax.axis_index("s")

        def fill_zero(i, _):
            def fill_lane(j, _):
                rows_a[i, pl.ds(j * 16, 16)] = jnp.zeros((16,), jnp.float32)
                return 0
            lax.fori_loop(0, D // 16, fill_lane, 0)
            return 0
        lax.fori_loop(0, CHUNK, fill_zero, 0)
        off = 0
        while off < rows_tile:
            zr = min(CHUNK, rows_tile - off)
            pltpu.sync_copy(
                rows_a.at[pl.ds(0, zr)],
                acc.at[pl.ds(s * rows_tile + off, zr)])
            off += zr
        plsc.subcore_barrier()

        n_pairs = (base_chunks + (1 if extra else 0)) // 2

        def body(i, _):
            base_a = c * E_sc + (s + (2 * i) * NS) * CHUNK
            base_b = c * E_sc + (s + (2 * i + 1) * NS) * CHUNK
            pltpu.sync_copy(src_hbm.at[pl.ds(base_a, CHUNK)], src_a)
            ga = pltpu.async_copy(g_hbm.at[src_a], rows_a, gsa)
            pltpu.sync_copy(src_hbm.at[pl.ds(base_b, CHUNK)], src_b)
            gb = pltpu.async_copy(g_hbm.at[src_b], rows_b, gsb)
            pltpu.sync_copy(dst_hbm.at[pl.ds(base_a, CHUNK)], dst_a)
            pltpu.sync_copy(dst_hbm.at[pl.ds(base_b, CHUNK)], dst_b)
            ga.wait()
            pltpu.sync_copy(rows_a, acc.at[dst_a], add=True)
            gb.wait()
            pltpu.sync_copy(rows_b, acc.at[dst_b], add=True)
            return 0
        lax.fori_loop(0, n_pairs, body, 0)
        plsc.subcore_barrier()

        pltpu.sync_copy(acc.at[pl.ds(s * rows_tile, rows_tile)],
                        out_hbm.at[pl.ds(c * Np + s * rows_tile, rows_tile)])

    return agg_kernel


def _dot(a, w):
    return lax.dot_general(a, w, (((1,), (0,)), ((), ())),
                           precision=lax.Precision.HIGHEST,
                           preferred_element_type=jnp.float32)


def _matmul_body(x_ref, w_ref, o_ref):
    o_ref[...] = _dot(x_ref[...], w_ref[...])


def _make_scale_body(N, Np):
    def body(deg_ref, h_ref, g_ref, dinv_ref):
        d = 1.0 + deg_ref[0:N, :] + deg_ref[Np:Np + N, :]
        dinv = lax.rsqrt(d)
        dinv_ref[...] = dinv
        g_ref[...] = h_ref[...] * dinv[:, 0:1]
    return body


def _make_mid_body(N, Np):
    def body(a_ref, g_ref, dinv_ref, b_ref, w_ref, o_ref):
        dinv = dinv_ref[...][:, 0:1]
        agg = a_ref[0:N, :] + a_ref[Np:Np + N, :] + g_ref[...]
        xn = jnp.maximum(agg * dinv + b_ref[...], 0.0)
        o_ref[...] = _dot(xn, w_ref[...]) * dinv
    return body


def _make_final_body(N, Np, D_out):
    def body(a_ref, g_ref, dinv_ref, b_ref, o_ref):
        dinv = dinv_ref[...][:, 0:1]
        agg = (a_ref[0:N, 0:D_out] + a_ref[Np:Np + N, 0:D_out]
               + g_ref[0:N, 0:D_out])
        o_ref[...] = agg * dinv + b_ref[...]
    return body


def kernel(x, edge_index, W1, b1, W2, b2, W3, b3):
    N, _ = x.shape
    E0 = edge_index.shape[1]
    grp = NC * NS * CHUNK
    E = -(-E0 // grp) * grp
    pad = E - E0
    rt = -(-(N // NS) // 8) * 8
    Npp = rt * NS
    pad_src = jnp.arange(pad, dtype=jnp.int32) % jnp.int32(N)
    src_p = jnp.concatenate([edge_index[0], pad_src])
    pad_dst = N + jnp.arange(pad, dtype=jnp.int32) % jnp.int32(Npp - N)
    dst_p = jnp.concatenate([edge_index[1], pad_dst])
    edge_index = jnp.stack([src_p, dst_p])
    D_hid = W1.shape[1]
    D_out = W3.shape[1]
    src = edge_index[0]
    dst = edge_index[1]

    f32 = jnp.float32
    Np = (-(-(N // NS) // 8) * 8) * NS
    deg_parts = _make_deg_kernel(E, N)(dst)

    h1 = pl.pallas_call(
        _matmul_body,
        out_shape=jax.ShapeDtypeStruct((N, D_hid), f32))(x, W1)

    g1, dinv16 = pl.pallas_call(
        _make_scale_body(N, Np),
        out_shape=[jax.ShapeDtypeStruct((N, D_hid), f32),
                   jax.ShapeDtypeStruct((N, DEG_W), f32)])(deg_parts, h1)

    agg_hid = _make_agg_kernel(E, N, D_hid)
    a1 = agg_hid(g1, src, dst)

    g2 = pl.pallas_call(
        _make_mid_body(N, Np),
        out_shape=jax.ShapeDtypeStruct((N, D_hid), f32))(
            a1, g1, dinv16, b1.reshape(1, -1), W2)

    a2 = agg_hid(g2, src, dst)

    W3p = jnp.pad(W3, ((0, 0), (0, D_hid - D_out)))
    g3 = pl.pallas_call(
        _make_mid_body(N, Np),
        out_shape=jax.ShapeDtypeStruct((N, D_hid), f32))(
            a2, g2, dinv16, b2.reshape(1, -1), W3p)

    a3 = agg_hid(g3, src, dst)

    out = pl.pallas_call(
        _make_final_body(N, Np, D_out),
        out_shape=jax.ShapeDtypeStruct((N, D_out), f32))(
            a3, g3, dinv16, b3.reshape(1, -1))

    return out
